# baseline (speedup 1.0000x reference)
import functools

import jax
import jax.numpy as jnp
from jax import lax
from jax.experimental import pallas as pl
from jax.experimental.pallas import tpu as pltpu
from jax.experimental.pallas import tpu_sc as plsc

_INFO = plsc.get_sparse_core_info()
_NC = _INFO.num_cores
_NS = _INFO.num_subcores
_NW = _NC * _NS

_CHUNK = 200
_NBUF = 4
_SKEW = 2


def _gather_body(n_chunks, table_hbm, idx_hbm, out_hbm,
                 idx_all, r0, r1, r2, r3, g0, g1, g2, g3, w0, w1, w2, w3):
    rows = (r0, r1, r2, r3)
    gsems = (g0, g1, g2, g3)
    wsems = (w0, w1, w2, w3)

    wid = lax.axis_index("s") * _NC + lax.axis_index("c")
    rows_per_w = n_chunks * _CHUNK
    base = wid * rows_per_w

    def out_slice(i):
        return out_hbm.at[pl.ds(pl.multiple_of(base + i * _CHUNK, 8), _CHUNK)]

    def idx_slice(i):
        return idx_all.at[pl.ds(pl.multiple_of(i * _CHUNK, 8), _CHUNK)]

    def start_gather(j, b):
        pltpu.async_copy(table_hbm.at[idx_slice(j)], rows[b], gsems[b])

    def wait_gather(j, b):
        pltpu.make_async_copy(table_hbm.at[idx_slice(j)], rows[b],
                              gsems[b]).wait()

    def start_write(i, b):
        pltpu.async_copy(rows[b], out_slice(i), wsems[b])

    def wait_write(i, b):
        pltpu.make_async_copy(rows[b], out_slice(i), wsems[b]).wait()

    def slot(i, b, first, last):
        wait_gather(i, b)
        start_write(i, b)
        bj = (b + _SKEW) % _NBUF
        if not first:
            wait_write(i - _SKEW, bj)
        if not last:
            start_gather(i + _SKEW, bj)

    pltpu.sync_copy(idx_hbm.at[pl.ds(pl.multiple_of(base, 8), rows_per_w)],
                    idx_all)
    for j in range(_SKEW):
        start_gather(j, j % _NBUF)

    for b in range(_NBUF):
        slot(b, b, first=b < _SKEW, last=False)

    def step(g, carry):
        for b in range(_NBUF):
            slot(g * _NBUF + b, b, first=False, last=False)
        return carry

    lax.fori_loop(1, n_chunks // _NBUF - 1, step, 0)

    for b in range(_NBUF):
        i = n_chunks - _NBUF + b
        slot(i, b, first=False, last=b >= _NBUF - _SKEW)

    for i in range(n_chunks - _SKEW, n_chunks):
        wait_write(i, i % _NBUF)


@functools.partial(jax.jit, static_argnames=("b", "l", "d"))
def _lookup(batch_flat, table, b, l, d):
    n = b * l
    assert n % (_NW * _CHUNK) == 0
    n_chunks = n // (_NW * _CHUNK)
    assert n_chunks % _NBUF == 0 and n_chunks >= 3 * _NBUF
    mesh = plsc.VectorSubcoreMesh(core_axis_name="c", subcore_axis_name="s")
    out = pl.kernel(
        functools.partial(_gather_body, n_chunks),
        out_type=jax.ShapeDtypeStruct((n, d), jnp.float32),
        mesh=mesh,
        scratch_types=[
            pltpu.VMEM((n // _NW,), jnp.int32),
            *([pltpu.VMEM((_CHUNK, d), jnp.float32)] * _NBUF),
            *([pltpu.SemaphoreType.DMA] * (2 * _NBUF)),
        ],
    )(table, batch_flat)
    return out.reshape(b, l, d)


def kernel(batch, table):
    b, l = batch.shape
    d = table.shape[1]
    return _lookup(batch.reshape(-1).astype(jnp.int32), table, b, l, d)


# speedup vs baseline: 1.0108x; 1.0108x over previous
import functools

import jax
import jax.numpy as jnp
from jax import lax
from jax.experimental import pallas as pl
from jax.experimental.pallas import tpu as pltpu
from jax.experimental.pallas import tpu_sc as plsc

_INFO = plsc.get_sparse_core_info()
_NC = _INFO.num_cores
_NS = _INFO.num_subcores
_NW = _NC * _NS

_NBUF = 4
_SKEW = 2


def _gather_body(n_chunks, chunk, table_hbm, idx_hbm, out_hbm,
                 i0, i1, i2, i3, r0, r1, r2, r3,
                 g0, g1, g2, g3, w0, w1, w2, w3):
    idxb = (i0, i1, i2, i3)
    rows = (r0, r1, r2, r3)
    gsems = (g0, g1, g2, g3)
    wsems = (w0, w1, w2, w3)

    wid = lax.axis_index("s") * _NC + lax.axis_index("c")
    base = wid * n_chunks

    def out_slice(i):
        return out_hbm.at[base + i]

    def start_gather(j, b):
        pltpu.sync_copy(idx_hbm.at[base + j], idxb[b])
        pltpu.async_copy(table_hbm.at[idxb[b]], rows[b], gsems[b])

    def wait_gather(j, b):
        pltpu.make_async_copy(table_hbm.at[idxb[b]], rows[b],
                              gsems[b]).wait()

    def start_write(i, b):
        pltpu.async_copy(rows[b], out_slice(i), wsems[b])

    def wait_write(i, b):
        pltpu.make_async_copy(rows[b], out_slice(i), wsems[b]).wait()

    def slot(i, b, first, last):
        wait_gather(i, b)
        start_write(i, b)
        bj = (b + _SKEW) % _NBUF
        if not first:
            wait_write(i - _SKEW, bj)
        if not last:
            start_gather(i + _SKEW, bj)

    for j in range(_SKEW):
        start_gather(j, j % _NBUF)

    for b in range(_NBUF):
        slot(b, b, first=b < _SKEW, last=False)

    def step(g, carry):
        for b in range(_NBUF):
            slot(g * _NBUF + b, b, first=False, last=False)
        return carry

    lax.fori_loop(1, n_chunks // _NBUF - 1, step, 0)

    for b in range(_NBUF):
        i = n_chunks - _NBUF + b
        slot(i, b, first=False, last=b >= _NBUF - _SKEW)

    for i in range(n_chunks - _SKEW, n_chunks):
        wait_write(i, i % _NBUF)


@functools.partial(jax.jit, static_argnames=("b", "l", "d"))
def _lookup(batch2d, table, b, l, d):
    assert b % _NW == 0
    n_chunks = b // _NW
    assert n_chunks % _NBUF == 0 and n_chunks >= 3 * _NBUF
    mesh = plsc.VectorSubcoreMesh(core_axis_name="c", subcore_axis_name="s")
    return pl.kernel(
        functools.partial(_gather_body, n_chunks, l),
        out_type=jax.ShapeDtypeStruct((b, l, d), jnp.float32),
        mesh=mesh,
        scratch_types=[
            *([pltpu.VMEM((l,), jnp.int32)] * _NBUF),
            *([pltpu.VMEM((l, d), jnp.float32)] * _NBUF),
            *([pltpu.SemaphoreType.DMA] * (2 * _NBUF)),
        ],
    )(table, batch2d)


def kernel(batch, table):
    b, l = batch.shape
    d = table.shape[1]
    if batch.dtype != jnp.int32:
        batch = batch.astype(jnp.int32)
    return _lookup(batch, table, b, l, d)


# speedup vs baseline: 1.0119x; 1.0010x over previous
import functools

import jax
import jax.numpy as jnp
from jax import lax
from jax.experimental import pallas as pl
from jax.experimental.pallas import tpu as pltpu
from jax.experimental.pallas import tpu_sc as plsc

_INFO = plsc.get_sparse_core_info()
_NC = _INFO.num_cores
_NS = _INFO.num_subcores
_NW = _NC * _NS

_NBUF = 4
_SKEW = 2


def _gather_body(n_chunks, chunk, table_hbm, idx_hbm, out_hbm,
                 i0, i1, i2, i3, r0, r1, r2, r3,
                 g0, g1, g2, g3, w0, w1, w2, w3):
    idxb = (i0, i1, i2, i3)
    rows = (r0, r1, r2, r3)
    gsems = (g0, g1, g2, g3)
    wsems = (w0, w1, w2, w3)

    wid = lax.axis_index("s") * _NC + lax.axis_index("c")
    base = wid * n_chunks

    def out_slice(i):
        return out_hbm.at[base + i]

    def start_gather(j, b):
        pltpu.sync_copy(idx_hbm.at[base + j], idxb[b])
        pltpu.async_copy(table_hbm.at[idxb[b]], rows[b], gsems[b])

    def wait_gather(j, b):
        pltpu.make_async_copy(table_hbm.at[idxb[b]], rows[b],
                              gsems[b]).wait()

    def start_write(i, b):
        pltpu.async_copy(rows[b], out_slice(i), wsems[b])

    def wait_write(i, b):
        pltpu.make_async_copy(rows[b], out_slice(i), wsems[b]).wait()

    def slot(i, b, first, last):
        wait_gather(i, b)
        start_write(i, b)
        bj = (b + _SKEW) % _NBUF
        if not first:
            wait_write(i - _SKEW, bj)
        if not last:
            start_gather(i + _SKEW, bj)

    for j in range(_SKEW):
        start_gather(j, j % _NBUF)

    for b in range(_NBUF):
        slot(b, b, first=b < _SKEW, last=False)

    def step(g, carry):
        for b in range(_NBUF):
            slot(g * _NBUF + b, b, first=False, last=False)
        return carry

    lax.fori_loop(1, n_chunks // _NBUF - 1, step, 0)

    for b in range(_NBUF):
        i = n_chunks - _NBUF + b
        slot(i, b, first=False, last=b >= _NBUF - _SKEW)

    for i in range(n_chunks - _SKEW, n_chunks):
        wait_write(i, i % _NBUF)


@functools.partial(jax.jit, static_argnames=("b", "l", "d"))
def _lookup(batch2d, table, b, l, d):
    assert b % _NW == 0
    n_chunks = b // _NW
    assert n_chunks % _NBUF == 0 and n_chunks >= 3 * _NBUF
    mesh = plsc.VectorSubcoreMesh(core_axis_name="c", subcore_axis_name="s")
    return pl.kernel(
        functools.partial(_gather_body, n_chunks, l),
        out_type=jax.ShapeDtypeStruct((b, l, d), jnp.float32),
        mesh=mesh,
        compiler_params=pltpu.CompilerParams(use_tc_tiling_on_sc=True),
        scratch_types=[
            *([pltpu.VMEM((l,), jnp.int32)] * _NBUF),
            *([pltpu.VMEM((l, d), jnp.float32)] * _NBUF),
            *([pltpu.SemaphoreType.DMA] * (2 * _NBUF)),
        ],
    )(table, batch2d)


def kernel(batch, table):
    b, l = batch.shape
    d = table.shape[1]
    if batch.dtype != jnp.int32:
        batch = batch.astype(jnp.int32)
    return _lookup(batch, table, b, l, d)
